# initial kernel scaffold (unmeasured)
import jax
import jax.numpy as jnp
from jax import lax
from jax.experimental import pallas as pl
from jax.experimental.pallas import tpu as pltpu

N_DEV = 16
SQ = 2048
SKV = 2048
H_PER = 8
DH = 128
D_MODEL = 1024
HD_PER = H_PER * DH
CHUNK = SQ // N_DEV
SCALE = 0.08838834764831843
Q_TILE = 1024


def _mod16(v):
    return lax.rem(v + 2 * N_DEV, N_DEV)


def _body(x_ref, wq_ref, k_ref, v_ref, wo_ref, out_ref,
          ctx_ref, rs_buf,
          rs_send, rs_recv, ag_send, ag_recv):
    my = lax.axis_index("i")
    left = _mod16(my - 1)
    right = _mod16(my + 1)

    barrier = pltpu.get_barrier_semaphore()
    for nbr in (left, right):
        pl.semaphore_signal(barrier, inc=1, device_id=(nbr,),
                            device_id_type=pl.DeviceIdType.MESH)
    pl.semaphore_wait(barrier, 2)

    q_all = (jnp.dot(x_ref[...], wq_ref[...],
                     preferred_element_type=jnp.float32)
             * SCALE).astype(jnp.bfloat16)

    for h in range(H_PER):
        k_h = k_ref[h]
        v_h = v_ref[h]
        for qt in range(SQ // Q_TILE):
            r0 = qt * Q_TILE
            q_blk = q_all[r0:r0 + Q_TILE, h * DH:(h + 1) * DH]
            s = lax.dot_general(
                q_blk, k_h, (((1,), (1,)), ((), ())),
                preferred_element_type=jnp.float32)
            qb = (lax.broadcasted_iota(jnp.int32, (Q_TILE, SKV), 0) + r0) // 64
            kb = lax.broadcasted_iota(jnp.int32, (Q_TILE, SKV), 1) // 64
            m = (qb == kb) | (kb == 0) | (((qb + kb) % 3) == 0)
            s = jnp.where(m, s, -1e9)
            mx = jnp.max(s, axis=1, keepdims=True)
            e = jnp.exp(s - mx)
            w = (e / jnp.sum(e, axis=1, keepdims=True)).astype(jnp.bfloat16)
            ctx_ref[r0:r0 + Q_TILE, h * DH:(h + 1) * DH] = jnp.dot(
                w, v_h, preferred_element_type=jnp.float32
            ).astype(jnp.bfloat16)

    out_ref[...] = jnp.dot(ctx_ref[...], wo_ref[...],
                           preferred_element_type=jnp.float32)

    for s_ in range(N_DEV - 1):
        send_c = _mod16(my - s_)
        rdma = pltpu.make_async_remote_copy(
            src_ref=out_ref.at[pl.ds(send_c * CHUNK, CHUNK), :],
            dst_ref=rs_buf.at[s_],
            send_sem=rs_send.at[s_],
            recv_sem=rs_recv.at[s_],
            device_id=(right,),
            device_id_type=pl.DeviceIdType.MESH,
        )
        rdma.start()
        rdma.wait()
        recv_c = _mod16(my - s_ - 1)
        rows = pl.ds(recv_c * CHUNK, CHUNK)
        out_ref[rows, :] = out_ref[rows, :] + rs_buf[s_]

    for t in range(N_DEV - 1):
        send_c = _mod16(my + 1 - t)
        rdma = pltpu.make_async_remote_copy(
            src_ref=out_ref.at[pl.ds(send_c * CHUNK, CHUNK), :],
            dst_ref=out_ref.at[pl.ds(send_c * CHUNK, CHUNK), :],
            send_sem=ag_send.at[t],
            recv_sem=ag_recv.at[t],
            device_id=(right,),
            device_id_type=pl.DeviceIdType.MESH,
        )
        rdma.start()
        rdma.wait()


def kernel(x, Wq, K_ext, V_ext, Wo):
    my = lax.axis_index("i")
    xb = x[0].astype(jnp.bfloat16)
    wq = lax.dynamic_slice_in_dim(Wq, my * HD_PER, HD_PER, 1
                                  ).astype(jnp.bfloat16)
    wo = lax.dynamic_slice_in_dim(Wo, my * HD_PER, HD_PER, 0
                                  ).astype(jnp.bfloat16)
    kb = K_ext[0].transpose(1, 0, 2).astype(jnp.bfloat16)
    vb = V_ext[0].transpose(1, 0, 2).astype(jnp.bfloat16)

    out2d = pl.pallas_call(
        _body,
        out_shape=jax.ShapeDtypeStruct((SQ, D_MODEL), jnp.float32),
        in_specs=[pl.BlockSpec(memory_space=pltpu.VMEM)] * 5,
        out_specs=pl.BlockSpec(memory_space=pltpu.VMEM),
        scratch_shapes=[
            pltpu.VMEM((SQ, HD_PER), jnp.bfloat16),
            pltpu.VMEM((N_DEV - 1, CHUNK, D_MODEL), jnp.float32),
            pltpu.SemaphoreType.DMA((N_DEV - 1,)),
            pltpu.SemaphoreType.DMA((N_DEV - 1,)),
            pltpu.SemaphoreType.DMA((N_DEV - 1,)),
            pltpu.SemaphoreType.DMA((N_DEV - 1,)),
        ],
        compiler_params=pltpu.CompilerParams(collective_id=0),
    )(xb, wq, kb, vb, wo)
    return out2d[None]


# baseline (device time: 336822 ns/iter reference)
import jax
import jax.numpy as jnp
from jax import lax
from jax.experimental import pallas as pl
from jax.experimental.pallas import tpu as pltpu

N_DEV = 16
SQ = 2048
SKV = 2048
H_PER = 8
DH = 128
D_MODEL = 1024
HD_PER = H_PER * DH
CHUNK = SQ // N_DEV
SCALE = 0.08838834764831843
Q_TILE = 512


def _mod16(v):
    return lax.rem(v + 2 * N_DEV, N_DEV)


def _body(x_ref, wq_ref, k_ref, v_ref, wo_ref, out_ref,
          ctx_ref, rs_buf,
          rs_send, rs_recv, ag_send, ag_recv):
    my = lax.axis_index("i")
    left = _mod16(my - 1)
    right = _mod16(my + 1)

    barrier = pltpu.get_barrier_semaphore()
    for nbr in (left, right):
        pl.semaphore_signal(barrier, inc=1, device_id=(nbr,),
                            device_id_type=pl.DeviceIdType.MESH)
    pl.semaphore_wait(barrier, 2)

    q_all = (jnp.dot(x_ref[...], wq_ref[...],
                     preferred_element_type=jnp.float32)
             * SCALE).astype(jnp.bfloat16)

    for qt in range(SQ // Q_TILE):
        r0 = qt * Q_TILE
        qb = (lax.broadcasted_iota(jnp.int32, (Q_TILE, SKV), 0) + r0) // 64
        kb = lax.broadcasted_iota(jnp.int32, (Q_TILE, SKV), 1) // 64
        m = (qb == kb) | (kb == 0) | (((qb + kb) % 3) == 0)
        bias = jnp.where(m, 0.0, -1e9).astype(jnp.float32)
        for h in range(H_PER):
            q_blk = q_all[r0:r0 + Q_TILE, h * DH:(h + 1) * DH]
            s = lax.dot_general(
                q_blk, k_ref[h], (((1,), (1,)), ((), ())),
                preferred_element_type=jnp.float32)
            s = s + bias
            mx = jnp.max(s, axis=1, keepdims=True)
            e = jnp.exp(s - mx)
            w = (e / jnp.sum(e, axis=1, keepdims=True)).astype(jnp.bfloat16)
            ctx_ref[r0:r0 + Q_TILE, h * DH:(h + 1) * DH] = jnp.dot(
                w, v_ref[h],
                preferred_element_type=jnp.float32
            ).astype(jnp.bfloat16)

    out_ref[...] = jnp.dot(ctx_ref[...], wo_ref[...],
                           preferred_element_type=jnp.float32)

    for s_ in range(N_DEV - 1):
        send_c = _mod16(my - s_)
        rdma = pltpu.make_async_remote_copy(
            src_ref=out_ref.at[pl.ds(send_c * CHUNK, CHUNK), :],
            dst_ref=rs_buf.at[s_],
            send_sem=rs_send.at[s_],
            recv_sem=rs_recv.at[s_],
            device_id=(right,),
            device_id_type=pl.DeviceIdType.MESH,
        )
        rdma.start()
        rdma.wait()
        recv_c = _mod16(my - s_ - 1)
        rows = pl.ds(recv_c * CHUNK, CHUNK)
        out_ref[rows, :] = out_ref[rows, :] + rs_buf[s_]

    for t in range(N_DEV - 1):
        send_c = _mod16(my + 1 - t)
        rdma = pltpu.make_async_remote_copy(
            src_ref=out_ref.at[pl.ds(send_c * CHUNK, CHUNK), :],
            dst_ref=out_ref.at[pl.ds(send_c * CHUNK, CHUNK), :],
            send_sem=ag_send.at[t],
            recv_sem=ag_recv.at[t],
            device_id=(right,),
            device_id_type=pl.DeviceIdType.MESH,
        )
        rdma.start()
        rdma.wait()


def kernel(x, Wq, K_ext, V_ext, Wo):
    my = lax.axis_index("i")
    xb = x[0].astype(jnp.bfloat16)
    wq = lax.dynamic_slice_in_dim(Wq, my * HD_PER, HD_PER, 1
                                  ).astype(jnp.bfloat16)
    wo = lax.dynamic_slice_in_dim(Wo, my * HD_PER, HD_PER, 0
                                  ).astype(jnp.bfloat16)
    kb = K_ext[0].transpose(1, 0, 2).astype(jnp.bfloat16)
    vb = V_ext[0].transpose(1, 0, 2).astype(jnp.bfloat16)

    out2d = pl.pallas_call(
        _body,
        out_shape=jax.ShapeDtypeStruct((SQ, D_MODEL), jnp.float32),
        in_specs=[pl.BlockSpec(memory_space=pltpu.VMEM)] * 5,
        out_specs=pl.BlockSpec(memory_space=pltpu.VMEM),
        scratch_shapes=[
            pltpu.VMEM((SQ, HD_PER), jnp.bfloat16),
            pltpu.VMEM((N_DEV - 1, CHUNK, D_MODEL), jnp.float32),
            pltpu.SemaphoreType.DMA((N_DEV - 1,)),
            pltpu.SemaphoreType.DMA((N_DEV - 1,)),
            pltpu.SemaphoreType.DMA((N_DEV - 1,)),
            pltpu.SemaphoreType.DMA((N_DEV - 1,)),
        ],
        compiler_params=pltpu.CompilerParams(
            collective_id=0, vmem_limit_bytes=38 * 1024 * 1024),
    )(xb, wq, kb, vb, wo)
    return out2d[None]


# device time: 180417 ns/iter; 1.8669x vs baseline; 1.8669x over previous
import jax
import jax.numpy as jnp
from jax import lax
from jax.experimental import pallas as pl
from jax.experimental.pallas import tpu as pltpu

N_DEV = 16
SQ = 2048
SKV = 2048
H_PER = 8
DH = 128
D_MODEL = 1024
HD_PER = H_PER * DH
CHUNK = SQ // N_DEV
SCALE = 0.08838834764831843
Q_TILE = 512


BITS = [[0, 2, 1, 3], [2, 0, 3, 1]]
COL = D_MODEL // 2


def _body(x_ref, wq_ref, k_ref, v_ref, wo_ref, out_ref,
          ctx_ref, sendb, rsb0, rsb1, rsb2, rsb3, agb,
          rs_send, rs_recv, ag_send, ag_recv):
    my = lax.axis_index("i")
    rs_bufs = [rsb0, rsb1, rsb2, rsb3]

    barrier = pltpu.get_barrier_semaphore()
    for k in range(4):
        pl.semaphore_signal(barrier, inc=1, device_id=(my ^ (1 << k),),
                            device_id_type=pl.DeviceIdType.MESH)
    pl.semaphore_wait(barrier, 4)

    q_all = (jnp.dot(x_ref[...], wq_ref[...],
                     preferred_element_type=jnp.float32)
             * SCALE).astype(jnp.bfloat16)

    for qt in range(SQ // Q_TILE):
        r0 = qt * Q_TILE
        qb = (lax.broadcasted_iota(jnp.int32, (Q_TILE, SKV), 0) + r0) // 64
        kb = lax.broadcasted_iota(jnp.int32, (Q_TILE, SKV), 1) // 64
        m = (qb == kb) | (kb == 0) | (((qb + kb) % 3) == 0)
        bias = jnp.where(m, 0.0, -1e9).astype(jnp.float32)
        for h in range(H_PER):
            q_blk = q_all[r0:r0 + Q_TILE, h * DH:(h + 1) * DH]
            s = lax.dot_general(
                q_blk, k_ref[h], (((1,), (1,)), ((), ())),
                preferred_element_type=jnp.float32)
            s = s + bias
            mx = jnp.max(s, axis=1, keepdims=True)
            e = jnp.exp(s - mx)
            w = (e / jnp.sum(e, axis=1, keepdims=True)).astype(jnp.bfloat16)
            ctx_ref[r0:r0 + Q_TILE, h * DH:(h + 1) * DH] = jnp.dot(
                w, v_ref[h],
                preferred_element_type=jnp.float32
            ).astype(jnp.bfloat16)

    out_ref[...] = jnp.dot(ctx_ref[...], wo_ref[...],
                           preferred_element_type=jnp.float32)

    starts = [0, 0]
    for j in range(4):
        sz = 1024 >> j
        rdmas = []
        for s in range(2):
            k = BITS[s][j]
            partner = my ^ (1 << k)
            keep_hi = (my >> k) & 1
            c0 = s * COL
            send_start = starts[s] + (1 - keep_hi) * sz
            starts[s] = starts[s] + keep_hi * sz
            sendb[0:sz, c0:c0 + COL] = out_ref[
                pl.ds(send_start, sz), c0:c0 + COL].astype(jnp.bfloat16)
            rdma = pltpu.make_async_remote_copy(
                src_ref=sendb.at[pl.ds(0, sz), pl.ds(c0, COL)],
                dst_ref=rs_bufs[j].at[:, pl.ds(c0, COL)],
                send_sem=rs_send.at[j, s],
                recv_sem=rs_recv.at[j, s],
                device_id=(partner,),
                device_id_type=pl.DeviceIdType.MESH,
            )
            rdma.start()
            rdmas.append(rdma)
        for s in range(2):
            rdmas[s].wait()
            c0 = s * COL
            rows = pl.ds(starts[s], sz)
            out_ref[rows, c0:c0 + COL] = (
                out_ref[rows, c0:c0 + COL]
                + rs_bufs[j][:, c0:c0 + COL].astype(jnp.float32))

    for s in range(2):
        c0 = s * COL
        agb[pl.ds(starts[s], CHUNK), c0:c0 + COL] = out_ref[
            pl.ds(starts[s], CHUNK), c0:c0 + COL].astype(jnp.bfloat16)

    cur = [starts[0], starts[1]]
    for j in range(4):
        sz = CHUNK << j
        rdmas = []
        for s in range(2):
            k = BITS[s][3 - j]
            partner = my ^ (1 << k)
            c0 = s * COL
            rdma = pltpu.make_async_remote_copy(
                src_ref=agb.at[pl.ds(cur[s], sz), pl.ds(c0, COL)],
                dst_ref=agb.at[pl.ds(cur[s], sz), pl.ds(c0, COL)],
                send_sem=ag_send.at[j, s],
                recv_sem=ag_recv.at[j, s],
                device_id=(partner,),
                device_id_type=pl.DeviceIdType.MESH,
            )
            rdma.start()
            rdmas.append(rdma)
        for s in range(2):
            rdmas[s].wait()
            k = BITS[s][3 - j]
            keep_hi = (my >> k) & 1
            cur[s] = cur[s] - keep_hi * sz

    out_ref[...] = agb[...].astype(jnp.float32)


def kernel(x, Wq, K_ext, V_ext, Wo):
    my = lax.axis_index("i")
    xb = x[0].astype(jnp.bfloat16)
    wq = lax.dynamic_slice_in_dim(Wq, my * HD_PER, HD_PER, 1
                                  ).astype(jnp.bfloat16)
    wo = lax.dynamic_slice_in_dim(Wo, my * HD_PER, HD_PER, 0
                                  ).astype(jnp.bfloat16)
    kb = K_ext[0].transpose(1, 0, 2).astype(jnp.bfloat16)
    vb = V_ext[0].transpose(1, 0, 2).astype(jnp.bfloat16)

    out2d = pl.pallas_call(
        _body,
        out_shape=jax.ShapeDtypeStruct((SQ, D_MODEL), jnp.float32),
        in_specs=[pl.BlockSpec(memory_space=pltpu.VMEM)] * 5,
        out_specs=pl.BlockSpec(memory_space=pltpu.VMEM),
        scratch_shapes=[
            pltpu.VMEM((SQ, HD_PER), jnp.bfloat16),
            pltpu.VMEM((1024, D_MODEL), jnp.bfloat16),
            pltpu.VMEM((1024, D_MODEL), jnp.bfloat16),
            pltpu.VMEM((512, D_MODEL), jnp.bfloat16),
            pltpu.VMEM((256, D_MODEL), jnp.bfloat16),
            pltpu.VMEM((128, D_MODEL), jnp.bfloat16),
            pltpu.VMEM((SQ, D_MODEL), jnp.bfloat16),
            pltpu.SemaphoreType.DMA((4, 2)),
            pltpu.SemaphoreType.DMA((4, 2)),
            pltpu.SemaphoreType.DMA((4, 2)),
            pltpu.SemaphoreType.DMA((4, 2)),
        ],
        compiler_params=pltpu.CompilerParams(
            collective_id=0, vmem_limit_bytes=40 * 1024 * 1024),
    )(xb, wq, kb, vb, wo)
    return out2d[None]


# device time: 161246 ns/iter; 2.0889x vs baseline; 1.1189x over previous
import jax
import jax.numpy as jnp
from jax import lax
from jax.experimental import pallas as pl
from jax.experimental.pallas import tpu as pltpu

N_DEV = 16
SQ = 2048
SKV = 2048
H_PER = 8
DH = 128
D_MODEL = 1024
HD_PER = H_PER * DH
CHUNK = SQ // N_DEV
SCALE = 0.08838834764831843
Q_TILE = 512


BITS = [[0, 2, 1, 3], [2, 0, 3, 1]]
COL = D_MODEL // 2


def _body(x_ref, wq_ref, k_ref, v_ref, wo_ref, out_ref,
          ctx_ref, sendb, rsb0, rsb1, rsb2, rsb3, agb,
          rs_send, rs_recv, ag_send, ag_recv):
    my = lax.axis_index("i")
    rs_bufs = [rsb0, rsb1, rsb2, rsb3]

    barrier = pltpu.get_barrier_semaphore()
    for k in range(4):
        pl.semaphore_signal(barrier, inc=1, device_id=(my ^ (1 << k),),
                            device_id_type=pl.DeviceIdType.MESH)
    pl.semaphore_wait(barrier, 4)

    q_all = (jnp.dot(x_ref[...], wq_ref[...],
                     preferred_element_type=jnp.float32)
             * SCALE).astype(jnp.bfloat16)

    for qt in range(SQ // Q_TILE):
        r0 = qt * Q_TILE
        qb = (lax.broadcasted_iota(jnp.int32, (Q_TILE, SKV), 0) + r0) // 64
        kb = lax.broadcasted_iota(jnp.int32, (Q_TILE, SKV), 1) // 64
        m = (qb == kb) | (kb == 0) | (((qb + kb) % 3) == 0)
        bias = jnp.where(m, 0.0, -1e9).astype(jnp.float32)
        for h in range(H_PER):
            q_blk = q_all[r0:r0 + Q_TILE, h * DH:(h + 1) * DH]
            s = lax.dot_general(
                q_blk, k_ref[h], (((1,), (1,)), ((), ())),
                preferred_element_type=jnp.float32)
            e = jnp.exp(s + bias)
            w = (e / jnp.sum(e, axis=1, keepdims=True)).astype(jnp.bfloat16)
            ctx_ref[r0:r0 + Q_TILE, h * DH:(h + 1) * DH] = jnp.dot(
                w, v_ref[h], preferred_element_type=jnp.float32
            ).astype(jnp.bfloat16)

    out_ref[...] = jnp.dot(ctx_ref[...], wo_ref[...],
                           preferred_element_type=jnp.float32)

    starts = [0, 0]
    for j in range(4):
        sz = 1024 >> j
        rdmas = []
        for s in range(2):
            k = BITS[s][j]
            partner = my ^ (1 << k)
            keep_hi = (my >> k) & 1
            c0 = s * COL
            send_start = starts[s] + (1 - keep_hi) * sz
            starts[s] = starts[s] + keep_hi * sz
            sendb[0:sz, c0:c0 + COL] = out_ref[
                pl.ds(send_start, sz), c0:c0 + COL].astype(jnp.bfloat16)
            rdma = pltpu.make_async_remote_copy(
                src_ref=sendb.at[pl.ds(0, sz), pl.ds(c0, COL)],
                dst_ref=rs_bufs[j].at[:, pl.ds(c0, COL)],
                send_sem=rs_send.at[j, s],
                recv_sem=rs_recv.at[j, s],
                device_id=(partner,),
                device_id_type=pl.DeviceIdType.MESH,
            )
            rdma.start()
            rdmas.append(rdma)
        for s in range(2):
            rdmas[s].wait()
            c0 = s * COL
            rows = pl.ds(starts[s], sz)
            out_ref[rows, c0:c0 + COL] = (
                out_ref[rows, c0:c0 + COL]
                + rs_bufs[j][:, c0:c0 + COL].astype(jnp.float32))

    for s in range(2):
        c0 = s * COL
        agb[pl.ds(starts[s], CHUNK), c0:c0 + COL] = out_ref[
            pl.ds(starts[s], CHUNK), c0:c0 + COL].astype(jnp.bfloat16)

    cur = [starts[0], starts[1]]
    for j in range(4):
        sz = CHUNK << j
        rdmas = []
        for s in range(2):
            k = BITS[s][3 - j]
            partner = my ^ (1 << k)
            c0 = s * COL
            rdma = pltpu.make_async_remote_copy(
                src_ref=agb.at[pl.ds(cur[s], sz), pl.ds(c0, COL)],
                dst_ref=agb.at[pl.ds(cur[s], sz), pl.ds(c0, COL)],
                send_sem=ag_send.at[j, s],
                recv_sem=ag_recv.at[j, s],
                device_id=(partner,),
                device_id_type=pl.DeviceIdType.MESH,
            )
            rdma.start()
            rdmas.append(rdma)
        for s in range(2):
            rdmas[s].wait()
            k = BITS[s][3 - j]
            keep_hi = (my >> k) & 1
            cur[s] = cur[s] - keep_hi * sz

    out_ref[...] = agb[...].astype(jnp.float32)


def kernel(x, Wq, K_ext, V_ext, Wo):
    my = lax.axis_index("i")
    xb = x[0].astype(jnp.bfloat16)
    wq = lax.dynamic_slice_in_dim(Wq, my * HD_PER, HD_PER, 1
                                  ).astype(jnp.bfloat16)
    wo = lax.dynamic_slice_in_dim(Wo, my * HD_PER, HD_PER, 0
                                  ).astype(jnp.bfloat16)
    kb = K_ext[0].transpose(1, 0, 2).astype(jnp.bfloat16)
    vb = V_ext[0].transpose(1, 0, 2).astype(jnp.bfloat16)

    out2d = pl.pallas_call(
        _body,
        out_shape=jax.ShapeDtypeStruct((SQ, D_MODEL), jnp.float32),
        in_specs=[pl.BlockSpec(memory_space=pltpu.VMEM)] * 5,
        out_specs=pl.BlockSpec(memory_space=pltpu.VMEM),
        scratch_shapes=[
            pltpu.VMEM((SQ, HD_PER), jnp.bfloat16),
            pltpu.VMEM((1024, D_MODEL), jnp.bfloat16),
            pltpu.VMEM((1024, D_MODEL), jnp.bfloat16),
            pltpu.VMEM((512, D_MODEL), jnp.bfloat16),
            pltpu.VMEM((256, D_MODEL), jnp.bfloat16),
            pltpu.VMEM((128, D_MODEL), jnp.bfloat16),
            pltpu.VMEM((SQ, D_MODEL), jnp.bfloat16),
            pltpu.SemaphoreType.DMA((4, 2)),
            pltpu.SemaphoreType.DMA((4, 2)),
            pltpu.SemaphoreType.DMA((4, 2)),
            pltpu.SemaphoreType.DMA((4, 2)),
        ],
        compiler_params=pltpu.CompilerParams(
            collective_id=0, vmem_limit_bytes=40 * 1024 * 1024),
    )(xb, wq, kb, vb, wo)
    return out2d[None]


# device time: 158455 ns/iter; 2.1257x vs baseline; 1.0176x over previous
import jax
import jax.numpy as jnp
from jax import lax
from jax.experimental import pallas as pl
from jax.experimental.pallas import tpu as pltpu

N_DEV = 16
SQ = 2048
SKV = 2048
H_PER = 8
DH = 128
D_MODEL = 1024
HD_PER = H_PER * DH
CHUNK = SQ // N_DEV
SCALE = 0.08838834764831843
Q_TILE = 512


BITS = [[0, 2, 1, 3], [2, 0, 3, 1]]
COL = D_MODEL // 2


def _body(x_ref, wq_ref, k_ref, v_ref, wo_ref, out_ref,
          ctx_ref, sendb, rsb0, rsb1, rsb2, rsb3, agb,
          rs_send, rs_recv, ag_send, ag_recv):
    my = lax.axis_index("i")
    rs_bufs = [rsb0, rsb1, rsb2, rsb3]

    barrier = pltpu.get_barrier_semaphore()
    for k in range(4):
        pl.semaphore_signal(barrier, inc=1, device_id=(my ^ (1 << k),),
                            device_id_type=pl.DeviceIdType.MESH)
    pl.semaphore_wait(barrier, 4)

    q_all = (jnp.dot(x_ref[...], wq_ref[...],
                     preferred_element_type=jnp.float32)
             * SCALE).astype(jnp.bfloat16)

    for qt in range(SQ // Q_TILE):
        r0 = qt * Q_TILE
        qb = (lax.broadcasted_iota(jnp.int32, (Q_TILE, SKV), 0) + r0) // 64
        kb = lax.broadcasted_iota(jnp.int32, (Q_TILE, SKV), 1) // 64
        m = (qb == kb) | (kb == 0) | (((qb + kb) % 3) == 0)
        bias = jnp.where(m, 0.0, -1e9).astype(jnp.float32)
        for h in range(H_PER):
            q_blk = q_all[r0:r0 + Q_TILE, h * DH:(h + 1) * DH]
            s = lax.dot_general(
                q_blk, k_ref[h], (((1,), (1,)), ((), ())),
                preferred_element_type=jnp.float32)
            e = jnp.exp(s + bias)
            w = (e / jnp.sum(e, axis=1, keepdims=True)).astype(jnp.bfloat16)
            ctx_ref[r0:r0 + Q_TILE, h * DH:(h + 1) * DH] = jnp.dot(
                w, v_ref[h], preferred_element_type=jnp.float32
            ).astype(jnp.bfloat16)

    def proj_quadrant(off, c0):
        out_ref[off:off + 1024, c0:c0 + COL] = jnp.dot(
            ctx_ref[off:off + 1024, :], wo_ref[:, c0:c0 + COL],
            preferred_element_type=jnp.float32)

    def j0_desc(s):
        c0 = s * COL
        return pltpu.make_async_remote_copy(
            src_ref=sendb.at[0:1024, pl.ds(c0, COL)],
            dst_ref=rs_bufs[0].at[:, pl.ds(c0, COL)],
            send_sem=rs_send.at[0, s],
            recv_sem=rs_recv.at[0, s],
            device_id=(my ^ (1 << BITS[s][0]),),
            device_id_type=pl.DeviceIdType.MESH,
        )

    bA = (my >> BITS[0][0]) & 1
    bB = (my >> BITS[1][0]) & 1
    for s, bit in ((0, bA), (1, bB)):
        for val in (0, 1):
            @pl.when(bit == val)
            def _(s=s, off=(1 - val) * 1024):
                c0 = s * COL
                proj_quadrant(off, c0)
                sendb[0:1024, c0:c0 + COL] = out_ref[
                    off:off + 1024, c0:c0 + COL].astype(jnp.bfloat16)
                j0_desc(s).start()
    for s, bit in ((0, bA), (1, bB)):
        for val in (0, 1):
            @pl.when(bit == val)
            def _(s=s, off=val * 1024):
                proj_quadrant(off, s * COL)

    starts = [bA * 1024, bB * 1024]
    for s in range(2):
        j0_desc(s).wait()
        c0 = s * COL
        rows = pl.ds(starts[s], 1024)
        out_ref[rows, c0:c0 + COL] = (
            out_ref[rows, c0:c0 + COL]
            + rs_bufs[0][:, c0:c0 + COL].astype(jnp.float32))

    for j in range(1, 4):
        sz = 1024 >> j
        rdmas = []
        for s in range(2):
            k = BITS[s][j]
            partner = my ^ (1 << k)
            keep_hi = (my >> k) & 1
            c0 = s * COL
            send_start = starts[s] + (1 - keep_hi) * sz
            starts[s] = starts[s] + keep_hi * sz
            sendb[0:sz, c0:c0 + COL] = out_ref[
                pl.ds(send_start, sz), c0:c0 + COL].astype(jnp.bfloat16)
            rdma = pltpu.make_async_remote_copy(
                src_ref=sendb.at[pl.ds(0, sz), pl.ds(c0, COL)],
                dst_ref=rs_bufs[j].at[:, pl.ds(c0, COL)],
                send_sem=rs_send.at[j, s],
                recv_sem=rs_recv.at[j, s],
                device_id=(partner,),
                device_id_type=pl.DeviceIdType.MESH,
            )
            rdma.start()
            rdmas.append(rdma)
        for s in range(2):
            rdmas[s].wait()
            c0 = s * COL
            rows = pl.ds(starts[s], sz)
            out_ref[rows, c0:c0 + COL] = (
                out_ref[rows, c0:c0 + COL]
                + rs_bufs[j][:, c0:c0 + COL].astype(jnp.float32))

    for s in range(2):
        c0 = s * COL
        agb[pl.ds(starts[s], CHUNK), c0:c0 + COL] = out_ref[
            pl.ds(starts[s], CHUNK), c0:c0 + COL].astype(jnp.bfloat16)

    cur = [starts[0], starts[1]]
    for j in range(4):
        sz = CHUNK << j
        rdmas = []
        for s in range(2):
            k = BITS[s][3 - j]
            partner = my ^ (1 << k)
            c0 = s * COL
            rdma = pltpu.make_async_remote_copy(
                src_ref=agb.at[pl.ds(cur[s], sz), pl.ds(c0, COL)],
                dst_ref=agb.at[pl.ds(cur[s], sz), pl.ds(c0, COL)],
                send_sem=ag_send.at[j, s],
                recv_sem=ag_recv.at[j, s],
                device_id=(partner,),
                device_id_type=pl.DeviceIdType.MESH,
            )
            rdma.start()
            rdmas.append(rdma)
        for s in range(2):
            rdmas[s].wait()
            k = BITS[s][3 - j]
            keep_hi = (my >> k) & 1
            cur[s] = cur[s] - keep_hi * sz

    out_ref[...] = agb[...].astype(jnp.float32)


def kernel(x, Wq, K_ext, V_ext, Wo):
    my = lax.axis_index("i")
    xb = x[0].astype(jnp.bfloat16)
    wq = lax.dynamic_slice_in_dim(Wq, my * HD_PER, HD_PER, 1
                                  ).astype(jnp.bfloat16)
    wo = lax.dynamic_slice_in_dim(Wo, my * HD_PER, HD_PER, 0
                                  ).astype(jnp.bfloat16)
    kb = K_ext[0].transpose(1, 0, 2).astype(jnp.bfloat16)
    vb = V_ext[0].transpose(1, 0, 2).astype(jnp.bfloat16)

    out2d = pl.pallas_call(
        _body,
        out_shape=jax.ShapeDtypeStruct((SQ, D_MODEL), jnp.float32),
        in_specs=[pl.BlockSpec(memory_space=pltpu.VMEM)] * 5,
        out_specs=pl.BlockSpec(memory_space=pltpu.VMEM),
        scratch_shapes=[
            pltpu.VMEM((SQ, HD_PER), jnp.bfloat16),
            pltpu.VMEM((1024, D_MODEL), jnp.bfloat16),
            pltpu.VMEM((1024, D_MODEL), jnp.bfloat16),
            pltpu.VMEM((512, D_MODEL), jnp.bfloat16),
            pltpu.VMEM((256, D_MODEL), jnp.bfloat16),
            pltpu.VMEM((128, D_MODEL), jnp.bfloat16),
            pltpu.VMEM((SQ, D_MODEL), jnp.bfloat16),
            pltpu.SemaphoreType.DMA((4, 2)),
            pltpu.SemaphoreType.DMA((4, 2)),
            pltpu.SemaphoreType.DMA((4, 2)),
            pltpu.SemaphoreType.DMA((4, 2)),
        ],
        compiler_params=pltpu.CompilerParams(
            collective_id=0, vmem_limit_bytes=40 * 1024 * 1024),
    )(xb, wq, kb, vb, wo)
    return out2d[None]
